# Initial kernel scaffold; baseline (speedup 1.0000x reference)
import jax
import jax.numpy as jnp
from jax.experimental import pallas as pl


def kernel(x, x_sim, edge_index_inter, edge_index_sim, W1i, b1i, W2i, b2i, W3i, b3i, W1s, b1s, W2s, b2s, W3s, b3s, Wpi, bpi, Wps, bps, Wf, bf):
    raise NotImplementedError("write your pallas kernel here")


# speedup vs baseline: 2.4917x; 2.4917x over previous
import functools

import jax
import jax.numpy as jnp
from jax import lax
from jax.experimental import pallas as pl
from jax.experimental.pallas import tpu as pltpu
from jax.experimental.pallas import tpu_sc as plsc

_N = 10000
_D = 128
_H = 256
_E = 320000
_TILES = 16
_CB = 128
_CHUNKS = 160
_EP = _TILES * _CHUNKS * _CB
_NP = 10240
_RPT = _NP // _TILES


_CSTG = 32
_GRP = _CHUNKS // _CSTG


def _make_agg():
    mesh = plsc.VectorSubcoreMesh(core_axis_name="c", subcore_axis_name="s")
    out_type = [
        jax.ShapeDtypeStruct((2, _NP, _D), jnp.float32),
        jax.ShapeDtypeStruct((2, _NP, _D), jnp.float32),
    ]
    scratch = [
        pltpu.VMEM((2, _CB), jnp.int32),
        pltpu.VMEM((2, _CB), jnp.int32),
        pltpu.VMEM((2, _CB), jnp.int32),
        pltpu.VMEM((2, _CB), jnp.int32),
        pltpu.VMEM((_CB, _D), jnp.float32),
        pltpu.VMEM((_CB, _D), jnp.float32),
        pltpu.VMEM_SHARED((_NP, _D), jnp.float32),
        pltpu.SemaphoreType.DMA,
        pltpu.SemaphoreType.DMA,
        pltpu.SemaphoreType.DMA,
        pltpu.SemaphoreType.DMA,
        pltpu.SemaphoreType.DMA,
    ]

    def body(tblI, tblS, idxI, idxS, z128, sumsI, sumsS,
             i0, i1, i2, i3, r0, r1, acc, semL, semG0, semG1,
             semS0, semS1):
        cid = lax.axis_index("c")
        sid = lax.axis_index("s")
        row0 = sid * _RPT
        idx = (i0, i1, i2, i3)
        rows = (r0, r1)
        semG = (semG0, semG1)
        semS = (semS0, semS1)
        for g in range(2):
            tbl = (tblI, tblS)[g]
            idx_h = (idxI, idxS)[g]
            sums = (sumsI, sumsS)[g]
            pltpu.sync_copy(z128, r0)
            for k in range(_RPT // _CB):
                pltpu.sync_copy(r0, acc.at[pl.ds(row0 + k * _CB, _CB)])
            plsc.subcore_barrier()

            def idx_l(j, u):
                return pltpu.make_async_copy(idx_h.at[cid, sid, j], idx[u],
                                             semL)

            def gat(u, ru):
                return pltpu.make_async_copy(tbl.at[idx[u].at[0]], rows[ru],
                                             semG[ru])

            def gat2(u, ru):
                hb = _CB // 2
                return (
                    pltpu.make_async_copy(
                        tbl.at[idx[u].at[0, pl.ds(0, hb)]],
                        rows[ru].at[pl.ds(0, hb)], semG[ru]),
                    pltpu.make_async_copy(
                        tbl.at[idx[u].at[0, pl.ds(hb, hb)]],
                        rows[ru].at[pl.ds(hb, hb)], semG[ru]),
                )

            idx_l(0, 0).start()
            idx_l(1, 1).start()
            idx_l(0, 0).wait()
            for d in gat2(0, 0):
                d.start()

            def emit_step(j, u, have_next, have_l2):
                if have_next:
                    un, rn = (u + 1) % 4, (u + 1) % 2
                    idx_l(j + 1, un).wait()
                    for d in gat2(un, rn):
                        d.start()
                if have_l2:
                    idx_l(j + 2, (u + 2) % 4).start()
                for d in gat2(u, u % 2):
                    d.wait()
                pltpu.sync_copy(rows[u % 2], acc.at[idx[u].at[1]], add=True)

            def quad(p, carry):
                j0 = 4 * p
                for u in range(4):
                    emit_step(j0 + u, u, True, True)
                return carry

            lax.fori_loop(0, _CHUNKS // 4 - 1, quad, 0)
            j0 = _CHUNKS - 4
            for u in range(4):
                j = j0 + u
                emit_step(j, u, have_next=(j + 1 < _CHUNKS),
                          have_l2=(j + 2 < _CHUNKS))
            plsc.subcore_barrier()
            for k in range(_RPT // _CB):
                sl = pl.ds(row0 + k * _CB, _CB)
                pltpu.sync_copy(acc.at[sl], r0)
                pltpu.sync_copy(r0, sums.at[cid, sl])

    return pl.kernel(body, out_type=out_type, mesh=mesh, scratch_types=scratch)


def _make_counts():
    mesh = plsc.VectorSubcoreMesh(core_axis_name="c", subcore_axis_name="s")

    def body(dst2, z128, o128, cnts, dst_v, buf, cacc, sem):
        cid = lax.axis_index("c")
        sid = lax.axis_index("s")
        row0 = sid * _RPT
        pltpu.sync_copy(z128, buf)
        for k in range(_RPT // _CB):
            pltpu.sync_copy(buf, cacc.at[pl.ds(row0 + k * _CB, _CB)])
        pltpu.sync_copy(o128, buf)
        plsc.subcore_barrier()
        for grp in range(_GRP):
            pltpu.sync_copy(dst2.at[cid, sid, pl.ds(grp * _CSTG, _CSTG)],
                            dst_v)

            def fire(j, carry):
                pltpu.async_copy(buf, cacc.at[dst_v.at[j]], sem, add=True)
                return carry

            def drain(j, carry):
                pltpu.make_async_copy(buf, cacc.at[dst_v.at[j]], sem).wait()
                return carry

            lax.fori_loop(0, _CSTG, fire, 0)
            lax.fori_loop(0, _CSTG, drain, 0)
        plsc.subcore_barrier()
        for k in range(_RPT // _CB):
            sl = pl.ds(row0 + k * _CB, _CB)
            pltpu.sync_copy(cacc.at[sl], buf)
            pltpu.sync_copy(buf, cnts.at[cid, sl])

    return pl.kernel(
        body,
        out_type=[jax.ShapeDtypeStruct((2, _NP, _CB), jnp.float32)],
        mesh=mesh,
        scratch_types=[
            pltpu.VMEM((_CSTG, _CB), jnp.int32),
            pltpu.VMEM((_CB, _CB), jnp.float32),
            pltpu.VMEM_SHARED((_NP, _CB), jnp.float32),
            pltpu.SemaphoreType.DMA,
        ],
    )


_R = 2000


def _mean_linear(s_ref, c_ref, w_ref, b_ref):
    inv = 1.0 / jnp.maximum(c_ref[:, 0:1], 1.0)
    h = jnp.dot(s_ref[0] * inv, w_ref[0:_D, :], preferred_element_type=jnp.float32)
    h = h + jnp.dot(s_ref[1] * inv, w_ref[_D:_H, :], preferred_element_type=jnp.float32)
    return h + b_ref[:]


def _layer_tc(sumsI, cntI, sumsS, cntS, WI, bI, WS, bS):
    def body(sI, cI, sS, cS, wI, bI_, wS, bS_, oI, oS):
        hI = jnp.maximum(_mean_linear(sI, cI, wI, bI_), 0.0)
        oI[0] = hI[:, 0:_D]
        oI[1] = hI[:, _D:_H]
        hS = jnp.maximum(_mean_linear(sS, cS, wS, bS_), 0.0)
        oS[0] = hS[:, 0:_D]
        oS[1] = hS[:, _D:_H]

    sum_spec = pl.BlockSpec((2, _R, _D), lambda i: (0, i, 0))
    cnt_spec = pl.BlockSpec((_R, 16), lambda i: (i, 0))
    w_spec = pl.BlockSpec((_H, _H), lambda i: (0, 0))
    b_spec = pl.BlockSpec((1, _H), lambda i: (0, 0))
    return pl.pallas_call(
        body,
        grid=(_N // _R,),
        in_specs=[sum_spec, cnt_spec, sum_spec, cnt_spec,
                  w_spec, b_spec, w_spec, b_spec],
        out_specs=[pl.BlockSpec((2, _R, _D), lambda i: (0, i, 0))] * 2,
        out_shape=[jax.ShapeDtypeStruct((2, _N, _D), jnp.float32)] * 2,
    )(sumsI, cntI, sumsS, cntS, WI, bI, WS, bS)


def _final_tc(sumsI, cntI, sumsS, cntS, W3i, b3i, W3s, b3s,
              Wpi, bpi, Wps, bps, Wf, bf):
    def body(sI, cI, sS, cS, w3i, b3i_, w3s, b3s_,
             wpi, bpi_, wps, bps_, wf, bf_, oh, ozi, ozs):
        hI = _mean_linear(sI, cI, w3i, b3i_)
        hS = _mean_linear(sS, cS, w3s, b3s_)
        ozi[...] = jnp.maximum(
            jnp.dot(hI, wpi[...], preferred_element_type=jnp.float32) + bpi_[:], 0.0)
        ozs[...] = jnp.maximum(
            jnp.dot(hS, wps[...], preferred_element_type=jnp.float32) + bps_[:], 0.0)
        oh[...] = jnp.dot(hI + hS, wf[...], preferred_element_type=jnp.float32) + bf_[:]

    sum_spec = pl.BlockSpec((2, _R, _D), lambda i: (0, i, 0))
    cnt_spec = pl.BlockSpec((_R, 16), lambda i: (i, 0))
    w_spec = pl.BlockSpec((_H, _H), lambda i: (0, 0))
    b_spec = pl.BlockSpec((1, _H), lambda i: (0, 0))
    return pl.pallas_call(
        body,
        grid=(_N // _R,),
        in_specs=[sum_spec, cnt_spec, sum_spec, cnt_spec,
                  w_spec, b_spec, w_spec, b_spec,
                  w_spec, b_spec, w_spec, b_spec,
                  pl.BlockSpec((_H, _D), lambda i: (0, 0)),
                  pl.BlockSpec((1, _D), lambda i: (0, 0))],
        out_specs=[pl.BlockSpec((_R, _D), lambda i: (i, 0)),
                   pl.BlockSpec((_R, _H), lambda i: (i, 0)),
                   pl.BlockSpec((_R, _H), lambda i: (i, 0))],
        out_shape=[jax.ShapeDtypeStruct((_N, _D), jnp.float32),
                   jax.ShapeDtypeStruct((_N, _H), jnp.float32),
                   jax.ShapeDtypeStruct((_N, _H), jnp.float32)],
    )(sumsI, cntI, sumsS, cntS, W3i, b3i, W3s, b3s,
      Wpi, bpi, Wps, bps, Wf, bf)


def _prep_edges(ei):
    src = jnp.pad(ei[0], (0, _EP - _E)).reshape(_TILES, _CHUNKS, _CB)
    dst = jnp.pad(ei[1], (0, _EP - _E),
                  constant_values=_N).reshape(_TILES, _CHUNKS, _CB)
    idx = jnp.stack([jnp.stack([src, dst], axis=2),
                     jnp.stack([src + _N, dst], axis=2)])
    return idx, dst


def kernel(x, x_sim, edge_index_inter, edge_index_sim,
           W1i, b1i, W2i, b2i, W3i, b3i,
           W1s, b1s, W2s, b2s, W3s, b3s,
           Wpi, bpi, Wps, bps, Wf, bf):
    tbl0 = jnp.concatenate([x, x_sim], axis=0)
    idxI, dstI = _prep_edges(edge_index_inter)
    idxS, dstS = _prep_edges(edge_index_sim)
    z128 = jnp.zeros((_CB, _D), jnp.float32)
    o128 = jnp.ones((_CB, _CB), jnp.float32)

    agg = _make_agg()
    counts = _make_counts()

    (cnts,) = counts(jnp.stack([dstI, dstS]), z128, o128)
    s1I, s1S = agg(tbl0, tbl0, idxI, idxS, z128)
    cntI, cntS = cnts[0, :, 0:16], cnts[1, :, 0:16]
    b = lambda v: v.reshape(1, -1)
    h1I, h1S = _layer_tc(s1I, cntI, s1S, cntS, W1i, b(b1i), W1s, b(b1s))
    s2I, s2S = agg(h1I.reshape(2 * _N, _D), h1S.reshape(2 * _N, _D),
                   idxI, idxS, z128)
    h2I, h2S = _layer_tc(s2I, cntI, s2S, cntS, W2i, b(b2i), W2s, b(b2s))
    s3I, s3S = agg(h2I.reshape(2 * _N, _D), h2S.reshape(2 * _N, _D),
                   idxI, idxS, z128)
    h, z_i, z_s = _final_tc(s3I, cntI, s3S, cntS, W3i, b(b3i), W3s, b(b3s),
                            Wpi, b(bpi), Wps, b(bps), Wf, b(bf))
    return (h, z_i, z_s)
